# baseline (speedup 1.0000x reference)
import jax
import jax.numpy as jnp
from jax.experimental import pallas as pl
from jax.experimental.pallas import tpu as pltpu

_B, _N, _DIN = 16, 1024, 128
_TM = 512
_NT = _N // _TM


def _lrelu(x):
    return jnp.where(x >= 0, x, x * 0.01)


def _pass0_body(g_ref, x_ref, w0_ref, b0_ref, h1_ref):
    t = jnp.dot(x_ref[0], w0_ref[...], preferred_element_type=jnp.float32)
    a = jnp.dot(g_ref[0], t.astype(jnp.bfloat16),
                preferred_element_type=jnp.float32)
    h1_ref[0] = _lrelu(a + b0_ref[...]).astype(jnp.bfloat16)


def _pass_mid_body(g_ref, h_ref, w_ref, b_ref, o_ref):
    a = jnp.dot(g_ref[0], h_ref[0], preferred_element_type=jnp.float32)
    z = jnp.dot(a.astype(jnp.bfloat16), w_ref[...],
                preferred_element_type=jnp.float32) + b_ref[...]
    o_ref[0] = _lrelu(z).astype(jnp.bfloat16)


def _pass3_body(g_ref, h_ref, w3_ref, b3_ref, wagg_ref, bagg_ref, pool_ref):
    a = jnp.dot(g_ref[0], h_ref[0], preferred_element_type=jnp.float32)
    h4 = _lrelu(jnp.dot(a.astype(jnp.bfloat16), w3_ref[...],
                        preferred_element_type=jnp.float32) + b3_ref[...])
    h5 = _lrelu(jnp.dot(h4.astype(jnp.bfloat16), wagg_ref[...],
                        preferred_element_type=jnp.float32) + bagg_ref[...])
    partial = jnp.sum(h5, axis=0, keepdims=True)[None]
    i = pl.program_id(1)

    @pl.when(i == 0)
    def _init():
        pool_ref[...] = partial

    @pl.when(i != 0)
    def _acc():
        pool_ref[...] += partial


def _head_body(p_ref, wfc_ref, bfc_ref, wout_ref, bout_ref, o_ref):
    z = _lrelu(jnp.dot(p_ref[...], wfc_ref[...],
                       preferred_element_type=jnp.float32) + bfc_ref[...])
    o_ref[...] = (jnp.dot(z, wout_ref[...], preferred_element_type=jnp.float32)
                  + bout_ref[...])


def _full(shape):
    return pl.BlockSpec(shape, lambda b, i: tuple(0 for _ in shape))


def _per_graph(shape):
    return pl.BlockSpec(shape, lambda b, i: (b,) + tuple(0 for _ in shape[1:]))


def _row_tile(shape):
    return pl.BlockSpec(shape, lambda b, i: (b, i) + tuple(0 for _ in shape[2:]))


def kernel(adj_list, feats, W_conv0, b_conv0, W_conv1, b_conv1, W_conv2,
           b_conv2, W_conv3, b_conv3, W_agg, b_agg, W_fc, b_fc, W_out, b_out):
    G = jnp.squeeze(adj_list, axis=-1).astype(jnp.bfloat16)
    b0 = jnp.reshape(b_conv0, (1, -1))
    b1 = jnp.reshape(b_conv1, (1, -1))
    b2 = jnp.reshape(b_conv2, (1, -1))
    b3 = jnp.reshape(b_conv3, (1, -1))
    bagg = jnp.reshape(b_agg, (1, -1))
    bfc = jnp.reshape(b_fc, (1, -1))
    bout = jnp.reshape(b_out, (1, -1))

    params = pltpu.CompilerParams(
        dimension_semantics=("parallel", "arbitrary"))

    h1 = pl.pallas_call(
        _pass0_body,
        grid=(_B, _NT),
        in_specs=[
            _row_tile((1, _TM, _N)),
            _per_graph((1, _N, _DIN)),
            _full(W_conv0.shape),
            _full(b0.shape),
        ],
        out_specs=_row_tile((1, _TM, 64)),
        out_shape=jax.ShapeDtypeStruct((_B, _N, 64), jnp.bfloat16),
        compiler_params=params,
    )(G, feats, W_conv0, b0)

    def mid(h, W, b, dout):
        return pl.pallas_call(
            _pass_mid_body,
            grid=(_B, _NT),
            in_specs=[
                _row_tile((1, _TM, _N)),
                _per_graph((1, _N, h.shape[-1])),
                _full(W.shape),
                _full(b.shape),
            ],
            out_specs=_row_tile((1, _TM, dout)),
            out_shape=jax.ShapeDtypeStruct((_B, _N, dout), jnp.bfloat16),
            compiler_params=params,
        )(G, h, W, b)

    h2 = mid(h1, W_conv1, b1, 64)
    h3 = mid(h2, W_conv2, b2, 128)

    pool = pl.pallas_call(
        _pass3_body,
        grid=(_B, _NT),
        in_specs=[
            _row_tile((1, _TM, _N)),
            _per_graph((1, _N, 128)),
            _full(W_conv3.shape),
            _full(b3.shape),
            _full(W_agg.shape),
            _full(bagg.shape),
        ],
        out_specs=pl.BlockSpec((1, 1, 72), lambda b, i: (b, 0, 0)),
        out_shape=jax.ShapeDtypeStruct((_B, 1, 72), jnp.float32),
        compiler_params=params,
    )(G, h3, W_conv3, b3, W_agg, bagg)
    pool = jnp.reshape(pool, (_B, 72))

    out = pl.pallas_call(
        _head_body,
        grid=(1,),
        in_specs=[
            pl.BlockSpec((_B, 72), lambda b: (0, 0)),
            pl.BlockSpec(W_fc.shape, lambda b: (0, 0)),
            pl.BlockSpec(bfc.shape, lambda b: (0, 0)),
            pl.BlockSpec(W_out.shape, lambda b: (0, 0)),
            pl.BlockSpec(bout.shape, lambda b: (0, 0)),
        ],
        out_specs=pl.BlockSpec((_B, 64), lambda b: (0, 0)),
        out_shape=jax.ShapeDtypeStruct((_B, 64), jnp.float32),
        compiler_params=pltpu.CompilerParams(
            dimension_semantics=("arbitrary",)),
    )(pool, W_fc, bfc, W_out, bout)

    return out


# speedup vs baseline: 1.3772x; 1.3772x over previous
import jax
import jax.numpy as jnp
from jax.experimental import pallas as pl
from jax.experimental.pallas import tpu as pltpu

_B, _N, _DIN = 16, 1024, 128
_GPB = 2
_NB = _B // _GPB


def _lrelu(x):
    return jnp.where(x >= 0, x, x * 0.01)


def _pass0_body(g_ref, x_ref, w0_ref, b0_ref, h1_ref):
    for j in range(_GPB):
        t = jnp.dot(x_ref[j], w0_ref[...], preferred_element_type=jnp.float32)
        a = jnp.dot(g_ref[j], t.astype(jnp.bfloat16),
                    preferred_element_type=jnp.float32)
        h1_ref[j] = _lrelu(a + b0_ref[...]).astype(jnp.bfloat16)


def _pass_mid_body(g_ref, h_ref, w_ref, b_ref, o_ref):
    for j in range(_GPB):
        a = jnp.dot(g_ref[j], h_ref[j], preferred_element_type=jnp.float32)
        z = jnp.dot(a.astype(jnp.bfloat16), w_ref[...],
                    preferred_element_type=jnp.float32) + b_ref[...]
        o_ref[j] = _lrelu(z).astype(jnp.bfloat16)


def _pass3_body(g_ref, h_ref, w3_ref, b3_ref, wagg_ref, bagg_ref, pool_ref):
    for j in range(_GPB):
        a = jnp.dot(g_ref[j], h_ref[j], preferred_element_type=jnp.float32)
        h4 = _lrelu(jnp.dot(a.astype(jnp.bfloat16), w3_ref[...],
                            preferred_element_type=jnp.float32) + b3_ref[...])
        h5 = _lrelu(jnp.dot(h4.astype(jnp.bfloat16), wagg_ref[...],
                            preferred_element_type=jnp.float32) + bagg_ref[...])
        pool_ref[j] = jnp.sum(h5, axis=0, keepdims=True)


def _head_body(p_ref, wfc_ref, bfc_ref, wout_ref, bout_ref, o_ref):
    z = _lrelu(jnp.dot(p_ref[...], wfc_ref[...],
                       preferred_element_type=jnp.float32) + bfc_ref[...])
    o_ref[...] = (jnp.dot(z, wout_ref[...], preferred_element_type=jnp.float32)
                  + bout_ref[...])


def _full(shape):
    return pl.BlockSpec(shape, lambda b: tuple(0 for _ in shape))


def _step_block(shape):
    return pl.BlockSpec(shape, lambda b: (b,) + tuple(0 for _ in shape[1:]))


def kernel(adj_list, feats, W_conv0, b_conv0, W_conv1, b_conv1, W_conv2,
           b_conv2, W_conv3, b_conv3, W_agg, b_agg, W_fc, b_fc, W_out, b_out):
    G = jnp.squeeze(adj_list, axis=-1).astype(jnp.bfloat16)
    b0 = jnp.reshape(b_conv0, (1, -1))
    b1 = jnp.reshape(b_conv1, (1, -1))
    b2 = jnp.reshape(b_conv2, (1, -1))
    b3 = jnp.reshape(b_conv3, (1, -1))
    bagg = jnp.reshape(b_agg, (1, -1))
    bfc = jnp.reshape(b_fc, (1, -1))
    bout = jnp.reshape(b_out, (1, -1))

    params = pltpu.CompilerParams(dimension_semantics=("arbitrary",))

    h1 = pl.pallas_call(
        _pass0_body,
        grid=(_NB,),
        in_specs=[
            _step_block((_GPB, _N, _N)),
            _step_block((_GPB, _N, _DIN)),
            _full(W_conv0.shape),
            _full(b0.shape),
        ],
        out_specs=_step_block((_GPB, _N, 64)),
        out_shape=jax.ShapeDtypeStruct((_B, _N, 64), jnp.bfloat16),
        compiler_params=params,
    )(G, feats, W_conv0, b0)

    def mid(h, W, b, dout):
        return pl.pallas_call(
            _pass_mid_body,
            grid=(_NB,),
            in_specs=[
                _step_block((_GPB, _N, _N)),
                _step_block((_GPB, _N, h.shape[-1])),
                _full(W.shape),
                _full(b.shape),
            ],
            out_specs=_step_block((_GPB, _N, dout)),
            out_shape=jax.ShapeDtypeStruct((_B, _N, dout), jnp.bfloat16),
            compiler_params=params,
        )(G, h, W, b)

    h2 = mid(h1, W_conv1, b1, 64)
    h3 = mid(h2, W_conv2, b2, 128)

    pool = pl.pallas_call(
        _pass3_body,
        grid=(_NB,),
        in_specs=[
            _step_block((_GPB, _N, _N)),
            _step_block((_GPB, _N, 128)),
            _full(W_conv3.shape),
            _full(b3.shape),
            _full(W_agg.shape),
            _full(bagg.shape),
        ],
        out_specs=_step_block((_GPB, 1, 72)),
        out_shape=jax.ShapeDtypeStruct((_B, 1, 72), jnp.float32),
        compiler_params=params,
    )(G, h3, W_conv3, b3, W_agg, bagg)
    pool = jnp.reshape(pool, (_B, 72))

    out = pl.pallas_call(
        _head_body,
        grid=(1,),
        in_specs=[
            _full((_B, 72)),
            _full(W_fc.shape),
            _full(bfc.shape),
            _full(W_out.shape),
            _full(bout.shape),
        ],
        out_specs=_full((_B, 64)),
        out_shape=jax.ShapeDtypeStruct((_B, 64), jnp.float32),
        compiler_params=params,
    )(pool, W_fc, bfc, W_out, bout)

    return out
